# baseline (speedup 1.0000x reference)
import itertools as it

import jax
import jax.numpy as jnp
from jax import lax
from jax.experimental import pallas as pl
from jax.experimental.pallas import tpu as pltpu
from jax.experimental.pallas import tpu_sc as plsc

DIM = 4
ROWS = 819200
ROW = 17
N = ROWS * ROW
NW = 32
ROWS_PER_W = ROWS // NW
PIECE_ROWS = 1600
PIECE = PIECE_ROWS * ROW
PIECES = ROWS_PER_W // PIECE_ROWS
PERIOD = 272
BLOCKS = PIECE // PERIOD


def _perm_idx():
    permutations = jnp.array(list(it.permutations(range(DIM))), dtype=jnp.int32)
    num_perms, num_ue = permutations.shape
    key = jax.random.key(42)
    _p = jax.random.randint(key, (1,), 0, num_perms, dtype=jnp.int32)
    perm = permutations[_p[0], :]
    t = jnp.tile(perm, num_ue)
    r = jnp.repeat(perm, num_ue, axis=0)
    idx = num_ue * r + t
    return jnp.concatenate((idx, jnp.array([num_ue ** 2], dtype=jnp.int32)))


def _permute_sc(x_flat, src0):
    mesh = plsc.VectorSubcoreMesh(core_axis_name="c", subcore_axis_name="s")

    @pl.kernel(
        out_type=jax.ShapeDtypeStruct((N,), jnp.float32),
        mesh=mesh,
        compiler_params=pltpu.CompilerParams(needs_layout_passes=False),
        scratch_types=[
            pltpu.VMEM((PIECE,), jnp.float32),
            pltpu.VMEM((PIECE,), jnp.float32),
            pltpu.VMEM((PERIOD,), jnp.int32),
        ],
    )
    def body(x_hbm, src_hbm, out_hbm, in_v, out_v, idx_v):
        wid = lax.axis_index("s") * 2 + lax.axis_index("c")
        woff = wid * (ROWS_PER_W * ROW)
        pltpu.sync_copy(src_hbm, idx_v)
        pats = [idx_v[pl.ds(j * 16, 16)] for j in range(ROW)]

        def piece_body(pi, _):
            off = woff + pi * PIECE
            pltpu.sync_copy(x_hbm.at[pl.ds(off, PIECE)], in_v)

            def blk(b, _):
                base = b * PERIOD
                for j in range(ROW):
                    idx = pats[j] + base
                    vals = plsc.load_gather(in_v, [idx])
                    out_v[pl.ds(base + j * 16, 16)] = vals
                return 0

            lax.fori_loop(0, BLOCKS, blk, 0)
            pltpu.sync_copy(out_v, out_hbm.at[pl.ds(off, PIECE)])
            return 0

        lax.fori_loop(0, PIECES, piece_body, 0)

    return body(x_flat, src0)


def kernel(x, training):
    perm_idx = _perm_idx()
    k = jnp.arange(PERIOD, dtype=jnp.int32)
    src_perm = (k // ROW) * ROW + perm_idx[k % ROW]
    src0 = jnp.where(training != 0, src_perm, k)
    out_flat = _permute_sc(x.reshape(N), src0)
    return out_flat.reshape(x.shape)


# speedup vs baseline: 1.0661x; 1.0661x over previous
import itertools as it

import jax
import jax.numpy as jnp
from jax import lax
from jax.experimental import pallas as pl
from jax.experimental.pallas import tpu as pltpu
from jax.experimental.pallas import tpu_sc as plsc

DIM = 4
ROWS = 819200
ROW = 17
N = ROWS * ROW
NW = 32
ROWS_PER_W = ROWS // NW
PIECE_ROWS = 1600
PIECE = PIECE_ROWS * ROW
PIECES = ROWS_PER_W // PIECE_ROWS
PERIOD = 272
UNROLL = 2


def _perm_idx():
    permutations = jnp.array(list(it.permutations(range(DIM))), dtype=jnp.int32)
    num_perms, num_ue = permutations.shape
    key = jax.random.key(42)
    _p = jax.random.randint(key, (1,), 0, num_perms, dtype=jnp.int32)
    perm = permutations[_p[0], :]
    t = jnp.tile(perm, num_ue)
    r = jnp.repeat(perm, num_ue, axis=0)
    idx = num_ue * r + t
    return jnp.concatenate((idx, jnp.array([num_ue ** 2], dtype=jnp.int32)))


def _permute_sc(x_flat, src0):
    mesh = plsc.VectorSubcoreMesh(core_axis_name="c", subcore_axis_name="s")

    @pl.kernel(
        out_type=jax.ShapeDtypeStruct((N,), jnp.float32),
        mesh=mesh,
        compiler_params=pltpu.CompilerParams(needs_layout_passes=False),
        scratch_types=[
            pltpu.VMEM((PIECE,), jnp.float32),
            pltpu.VMEM((PIECE,), jnp.float32),
            pltpu.VMEM((PIECE,), jnp.float32),
            pltpu.VMEM((PIECE,), jnp.float32),
            pltpu.VMEM((PERIOD,), jnp.int32),
            pltpu.SemaphoreType.DMA,
            pltpu.SemaphoreType.DMA,
            pltpu.SemaphoreType.DMA,
            pltpu.SemaphoreType.DMA,
        ],
    )
    def body(x_hbm, src_hbm, out_hbm, in0, in1, o0, o1, idx_v,
             isem0, isem1, osem0, osem1):
        wid = lax.axis_index("s") * 2 + lax.axis_index("c")
        woff = wid * (ROWS_PER_W * ROW)
        pltpu.sync_copy(src_hbm, idx_v)
        pats = [idx_v[pl.ds(j * 16, 16)] for j in range(ROW)]
        ins, outs = (in0, in1), (o0, o1)
        isems, osems = (isem0, isem1), (osem0, osem1)

        def in_copy(p):
            b = p % 2
            return pltpu.make_async_copy(
                x_hbm.at[pl.ds(woff + p * PIECE, PIECE)], ins[b], isems[b])

        def out_copy(p):
            b = p % 2
            return pltpu.make_async_copy(
                outs[b], out_hbm.at[pl.ds(woff + p * PIECE, PIECE)], osems[b])

        in_copy(0).start()
        for p in range(PIECES):
            b = p % 2
            in_copy(p).wait()
            if p + 1 < PIECES:
                in_copy(p + 1).start()
            if p >= 2:
                out_copy(p - 2).wait()
            in_b, out_b = ins[b], outs[b]

            @plsc.parallel_loop(0, PIECE, PERIOD, unroll=UNROLL)
            def blk(base):
                view = in_b.at[pl.ds(base, PERIOD)]
                for j in range(ROW):
                    out_b[pl.ds(base + j * 16, 16)] = plsc.load_gather(
                        view, [pats[j]])

            out_copy(p).start()
        out_copy(PIECES - 2).wait()
        out_copy(PIECES - 1).wait()

    return body(x_flat, src0)


def kernel(x, training):
    perm_idx = _perm_idx()
    k = jnp.arange(PERIOD, dtype=jnp.int32)
    src_perm = (k // ROW) * ROW + perm_idx[k % ROW]
    src0 = jnp.where(training != 0, src_perm, k)
    out_flat = _permute_sc(x.reshape(N), src0)
    return out_flat.reshape(x.shape)


# speedup vs baseline: 2.2384x; 2.0996x over previous
import itertools as it

import jax
import jax.numpy as jnp
from jax import lax
from jax.experimental import pallas as pl
from jax.experimental.pallas import tpu as pltpu
from jax.experimental.pallas import tpu_sc as plsc

DIM = 4
ROWS = 819200
ROW = 17
N = ROWS * ROW
NW = 32
ROWS_PER_W = ROWS // NW
PIECE_ROWS = 1600
PIECE = PIECE_ROWS * ROW
PIECES = ROWS_PER_W // PIECE_ROWS
PERIOD = 272
UNROLL = 2


def _perm_idx():
    permutations = jnp.array(list(it.permutations(range(DIM))), dtype=jnp.int32)
    num_perms, num_ue = permutations.shape
    key = jax.random.key(42)
    _p = jax.random.randint(key, (1,), 0, num_perms, dtype=jnp.int32)
    perm = permutations[_p[0], :]
    t = jnp.tile(perm, num_ue)
    r = jnp.repeat(perm, num_ue, axis=0)
    idx = num_ue * r + t
    return jnp.concatenate((idx, jnp.array([num_ue ** 2], dtype=jnp.int32)))


def _permute_sc(x_flat, src0):
    mesh = plsc.VectorSubcoreMesh(core_axis_name="c", subcore_axis_name="s")

    @pl.kernel(
        out_type=jax.ShapeDtypeStruct((N,), jnp.float32),
        mesh=mesh,
        compiler_params=pltpu.CompilerParams(
            needs_layout_passes=False, use_tc_tiling_on_sc=True),
        scratch_types=[
            pltpu.VMEM((PIECE,), jnp.float32),
            pltpu.VMEM((PIECE,), jnp.float32),
            pltpu.VMEM((PIECE,), jnp.float32),
            pltpu.VMEM((PIECE,), jnp.float32),
            pltpu.VMEM((PERIOD,), jnp.int32),
            pltpu.SemaphoreType.DMA,
            pltpu.SemaphoreType.DMA,
            pltpu.SemaphoreType.DMA,
            pltpu.SemaphoreType.DMA,
        ],
    )
    def body(x_hbm, src_hbm, out_hbm, in0, in1, o0, o1, idx_v,
             isem0, isem1, osem0, osem1):
        wid = lax.axis_index("s") * 2 + lax.axis_index("c")
        woff = wid * (ROWS_PER_W * ROW)
        pltpu.sync_copy(src_hbm, idx_v)
        pats = [idx_v[pl.ds(j * 16, 16)] for j in range(ROW)]
        ins, outs = (in0, in1), (o0, o1)
        isems, osems = (isem0, isem1), (osem0, osem1)

        def in_copy(p):
            b = p % 2
            return pltpu.make_async_copy(
                x_hbm.at[pl.ds(woff + p * PIECE, PIECE)], ins[b], isems[b])

        def out_copy(p):
            b = p % 2
            return pltpu.make_async_copy(
                outs[b], out_hbm.at[pl.ds(woff + p * PIECE, PIECE)], osems[b])

        in_copy(0).start()
        for p in range(PIECES):
            b = p % 2
            in_copy(p).wait()
            if p + 1 < PIECES:
                in_copy(p + 1).start()
            if p >= 2:
                out_copy(p - 2).wait()
            in_b, out_b = ins[b], outs[b]

            @plsc.parallel_loop(0, PIECE, PERIOD, unroll=UNROLL)
            def blk(base):
                view = in_b.at[pl.ds(base, PERIOD)]
                for j in range(ROW):
                    out_b[pl.ds(base + j * 16, 16)] = plsc.load_gather(
                        view, [pats[j]])

            out_copy(p).start()
        out_copy(PIECES - 2).wait()
        out_copy(PIECES - 1).wait()

    return body(x_flat, src0)


BATCH_BLK = 64


def _permute_tc(x, p_mat):
    grid = x.shape[0] // BATCH_BLK

    def body(x_ref, p_ref, o_ref):
        blk = x_ref[...].reshape(BATCH_BLK * 200, ROW)
        out = jax.lax.dot_general(
            blk, p_ref[...], (((1,), (0,)), ((), ())),
            precision=jax.lax.Precision.HIGHEST,
            preferred_element_type=jnp.float32)
        o_ref[...] = out.reshape(BATCH_BLK, 200, ROW)

    return pl.pallas_call(
        body,
        grid=(grid,),
        in_specs=[
            pl.BlockSpec((BATCH_BLK, 200, ROW), lambda i: (i, 0, 0)),
            pl.BlockSpec((ROW, ROW), lambda i: (0, 0)),
        ],
        out_specs=pl.BlockSpec((BATCH_BLK, 200, ROW), lambda i: (i, 0, 0)),
        out_shape=jax.ShapeDtypeStruct(x.shape, jnp.float32),
    )(x, p_mat)


def kernel(x, training):
    perm_idx = _perm_idx()
    idx_eff = jnp.where(training != 0, perm_idx,
                        jnp.arange(ROW, dtype=jnp.int32))
    p_mat = (idx_eff[None, :] == jnp.arange(ROW, dtype=jnp.int32)[:, None]
             ).astype(jnp.float32)
    return _permute_tc(x, p_mat)


# speedup vs baseline: 2.4648x; 1.1011x over previous
import itertools as it

import jax
import jax.numpy as jnp
from jax import lax
from jax.experimental import pallas as pl
from jax.experimental.pallas import tpu as pltpu
from jax.experimental.pallas import tpu_sc as plsc

DIM = 4
ROWS = 819200
ROW = 17
N = ROWS * ROW
NW = 32
ROWS_PER_W = ROWS // NW
PIECE_ROWS = 1600
PIECE = PIECE_ROWS * ROW
PIECES = ROWS_PER_W // PIECE_ROWS
PERIOD = 272
UNROLL = 2


def _perm_idx():
    permutations = jnp.array(list(it.permutations(range(DIM))), dtype=jnp.int32)
    num_perms, num_ue = permutations.shape
    key = jax.random.key(42)
    _p = jax.random.randint(key, (1,), 0, num_perms, dtype=jnp.int32)
    perm = permutations[_p[0], :]
    t = jnp.tile(perm, num_ue)
    r = jnp.repeat(perm, num_ue, axis=0)
    idx = num_ue * r + t
    return jnp.concatenate((idx, jnp.array([num_ue ** 2], dtype=jnp.int32)))


def _permute_sc(x_flat, src0):
    mesh = plsc.VectorSubcoreMesh(core_axis_name="c", subcore_axis_name="s")

    @pl.kernel(
        out_type=jax.ShapeDtypeStruct((N,), jnp.float32),
        mesh=mesh,
        compiler_params=pltpu.CompilerParams(
            needs_layout_passes=False, use_tc_tiling_on_sc=True),
        scratch_types=[
            pltpu.VMEM((PIECE,), jnp.float32),
            pltpu.VMEM((PIECE,), jnp.float32),
            pltpu.VMEM((PIECE,), jnp.float32),
            pltpu.VMEM((PIECE,), jnp.float32),
            pltpu.VMEM((PERIOD,), jnp.int32),
            pltpu.SemaphoreType.DMA,
            pltpu.SemaphoreType.DMA,
            pltpu.SemaphoreType.DMA,
            pltpu.SemaphoreType.DMA,
        ],
    )
    def body(x_hbm, src_hbm, out_hbm, in0, in1, o0, o1, idx_v,
             isem0, isem1, osem0, osem1):
        wid = lax.axis_index("s") * 2 + lax.axis_index("c")
        woff = wid * (ROWS_PER_W * ROW)
        pltpu.sync_copy(src_hbm, idx_v)
        pats = [idx_v[pl.ds(j * 16, 16)] for j in range(ROW)]
        ins, outs = (in0, in1), (o0, o1)
        isems, osems = (isem0, isem1), (osem0, osem1)

        def in_copy(p):
            b = p % 2
            return pltpu.make_async_copy(
                x_hbm.at[pl.ds(woff + p * PIECE, PIECE)], ins[b], isems[b])

        def out_copy(p):
            b = p % 2
            return pltpu.make_async_copy(
                outs[b], out_hbm.at[pl.ds(woff + p * PIECE, PIECE)], osems[b])

        in_copy(0).start()
        for p in range(PIECES):
            b = p % 2
            in_copy(p).wait()
            if p + 1 < PIECES:
                in_copy(p + 1).start()
            if p >= 2:
                out_copy(p - 2).wait()
            in_b, out_b = ins[b], outs[b]

            @plsc.parallel_loop(0, PIECE, PERIOD, unroll=UNROLL)
            def blk(base):
                view = in_b.at[pl.ds(base, PERIOD)]
                for j in range(ROW):
                    out_b[pl.ds(base + j * 16, 16)] = plsc.load_gather(
                        view, [pats[j]])

            out_copy(p).start()
        out_copy(PIECES - 2).wait()
        out_copy(PIECES - 1).wait()

    return body(x_flat, src0)


BATCH_BLK = 64


def _permute_tc(x, p_mat):
    grid = x.shape[0] // BATCH_BLK

    def body(x_ref, idx_ref, o_ref):
        idx = jnp.broadcast_to(idx_ref[...][None, None, :],
                               (BATCH_BLK, 200, ROW))
        o_ref[...] = jnp.take_along_axis(x_ref[...], idx, axis=-1)

    return pl.pallas_call(
        body,
        grid=(grid,),
        in_specs=[
            pl.BlockSpec((BATCH_BLK, 200, ROW), lambda i: (i, 0, 0)),
            pl.BlockSpec((ROW,), lambda i: (0,)),
        ],
        out_specs=pl.BlockSpec((BATCH_BLK, 200, ROW), lambda i: (i, 0, 0)),
        out_shape=jax.ShapeDtypeStruct(x.shape, jnp.float32),
    )(x, p_mat)


def kernel(x, training):
    perm_idx = _perm_idx()
    idx_eff = jnp.where(training != 0, perm_idx,
                        jnp.arange(ROW, dtype=jnp.int32))
    return _permute_tc(x, idx_eff)
